# baseline (speedup 1.0000x reference)
import jax
import jax.numpy as jnp
from jax.experimental import pallas as pl
from jax.experimental.pallas import tpu as pltpu


_COLS = 128
_ROWS = 12500
_SIZES = (512, 1024, 2048, 2560, 2560, 2560, 1236)
_STEP = max(_SIZES)
_NB = 4


def _steps():
    out = []
    off = 0
    for nr in _SIZES:
        out.append((off, nr))
        off += nr
    assert off == _ROWS
    return out


def _body(p_hbm, a_hbm, b_hbm, o_hbm, p_v, a_v, b_v, o_v, in_sem, out_sem):
    steps = _steps()
    ns = len(steps)

    def in_copies(s):
        off, nr = steps[s]
        b = s % _NB
        h = (nr // 2) & ~7
        cps = []
        for src, dst in ((p_hbm, p_v), (a_hbm, a_v), (b_hbm, b_v)):
            cps.append(pltpu.make_async_copy(
                src.at[pl.ds(off, h)], dst.at[b, pl.ds(0, h)], in_sem.at[b]))
            cps.append(pltpu.make_async_copy(
                src.at[pl.ds(off + h, nr - h)],
                dst.at[b, pl.ds(h, nr - h)], in_sem.at[b]))
        return cps

    def out_copy(s):
        off, nr = steps[s]
        b = s % _NB
        return pltpu.make_async_copy(
            o_v.at[b, pl.ds(0, nr)], o_hbm.at[pl.ds(off, nr)], out_sem.at[b])

    for s in range(min(_NB - 1, ns)):
        for cp in in_copies(s):
            cp.start()

    for s in range(ns):
        b = s % _NB
        if s + _NB - 1 < ns:
            for cp in in_copies(s + _NB - 1):
                cp.start()
        for cp in in_copies(s):
            cp.wait()
        if s >= _NB:
            out_copy(s - _NB).wait()
        nr = steps[s][1]
        o_v[b, :nr, :] = (
            a_v[b, :nr, :]
            + b_v[b, :nr, :] * jnp.maximum(p_v[b, :nr, :], 0.0))
        out_copy(s).start()

    for s in range(max(0, ns - _NB), ns):
        out_copy(s).wait()


def kernel(products, A, B):
    p2 = products.reshape(_ROWS, _COLS)
    a2 = A.reshape(_ROWS, _COLS)
    b2 = B.reshape(_ROWS, _COLS)
    hbm = pl.BlockSpec(memory_space=pl.ANY)
    out = pl.pallas_call(
        _body,
        in_specs=[hbm, hbm, hbm],
        out_specs=hbm,
        out_shape=jax.ShapeDtypeStruct((_ROWS, _COLS), jnp.float32),
        scratch_shapes=[
            pltpu.VMEM((_NB, _STEP, _COLS), jnp.float32),
            pltpu.VMEM((_NB, _STEP, _COLS), jnp.float32),
            pltpu.VMEM((_NB, _STEP, _COLS), jnp.float32),
            pltpu.VMEM((_NB, _STEP, _COLS), jnp.float32),
            pltpu.SemaphoreType.DMA((_NB,)),
            pltpu.SemaphoreType.DMA((_NB,)),
        ],
    )(p2, a2, b2)
    return out.reshape(_ROWS * _COLS)


# speedup vs baseline: 1.0079x; 1.0079x over previous
import jax
import jax.numpy as jnp
from jax.experimental import pallas as pl
from jax.experimental.pallas import tpu as pltpu


_COLS = 128
_ROWS = 12500
_SIZES = (512, 1024, 2048, 2560, 2560, 2560, 1236)
_STEP = max(_SIZES)
_NB = 3


def _steps():
    out = []
    off = 0
    for nr in _SIZES:
        out.append((off, nr))
        off += nr
    assert off == _ROWS
    return out


def _body(p_hbm, a_hbm, b_hbm, o_hbm, p_v, a_v, b_v, o_v, in_sem, out_sem):
    steps = _steps()
    ns = len(steps)

    def in_copies(s):
        off, nr = steps[s]
        b = s % _NB
        sl = pl.ds(off, nr)
        vsl = pl.ds(0, nr)
        return [
            pltpu.make_async_copy(p_hbm.at[sl], p_v.at[b, vsl], in_sem.at[b]),
            pltpu.make_async_copy(a_hbm.at[sl], a_v.at[b, vsl], in_sem.at[b]),
            pltpu.make_async_copy(b_hbm.at[sl], b_v.at[b, vsl], in_sem.at[b]),
        ]

    def out_copy(s):
        off, nr = steps[s]
        b = s % _NB
        return pltpu.make_async_copy(
            o_v.at[b, pl.ds(0, nr)], o_hbm.at[pl.ds(off, nr)], out_sem.at[b])

    for s in range(min(_NB - 1, ns)):
        for cp in in_copies(s):
            cp.start()

    for s in range(ns):
        b = s % _NB
        if s + _NB - 1 < ns:
            for cp in in_copies(s + _NB - 1):
                cp.start()
        for cp in in_copies(s):
            cp.wait()
        if s >= _NB:
            out_copy(s - _NB).wait()
        nr = steps[s][1]
        o_v[b, :nr, :] = (
            a_v[b, :nr, :]
            + b_v[b, :nr, :] * jnp.maximum(p_v[b, :nr, :], 0.0))
        out_copy(s).start()

    for s in range(max(0, ns - _NB), ns):
        out_copy(s).wait()


def kernel(products, A, B):
    p2 = products.reshape(_ROWS, _COLS)
    a2 = A.reshape(_ROWS, _COLS)
    b2 = B.reshape(_ROWS, _COLS)
    hbm = pl.BlockSpec(memory_space=pl.ANY)
    out = pl.pallas_call(
        _body,
        in_specs=[hbm, hbm, hbm],
        out_specs=hbm,
        out_shape=jax.ShapeDtypeStruct((_ROWS, _COLS), jnp.float32),
        scratch_shapes=[
            pltpu.VMEM((_NB, _STEP, _COLS), jnp.float32),
            pltpu.VMEM((_NB, _STEP, _COLS), jnp.float32),
            pltpu.VMEM((_NB, _STEP, _COLS), jnp.float32),
            pltpu.VMEM((_NB, _STEP, _COLS), jnp.float32),
            pltpu.SemaphoreType.DMA((_NB,)),
            pltpu.SemaphoreType.DMA((_NB,)),
        ],
    )(p2, a2, b2)
    return out.reshape(_ROWS * _COLS)
